# baseline (speedup 1.0000x reference)
import dataclasses
import functools

import jax
import jax.numpy as jnp
from jax import lax
from jax.experimental import pallas as pl
from jax.experimental.pallas import tpu as pltpu
from jax.experimental.pallas import tpu_sc as plsc

N_ATOMS = 10000
N_EDGES = 320000
D = 128
N_RBF = 20

NC = 2
NS = 16
NW = NC * NS
CHUNK = 128
NCH_TOTAL = N_EDGES // CHUNK
SPLIT = N_ATOMS // NC
ACC_ROWS = 5120
SLAB = ACC_ROWS // NS
TMAX = 160
IR = 4


_LOG2 = 0.6931471805599453


def _ssp(t):
    return jnp.maximum(t, 0.0) + jnp.log(0.5 * (1.0 + jnp.exp(-jnp.abs(t))))


def _bdot(a, b):
    return jnp.dot(a.astype(jnp.bfloat16), b.astype(jnp.bfloat16),
                   preferred_element_type=jnp.float32)


def _h_body(x_ref, w_ref, o_ref):
    o_ref[...] = _bdot(x_ref[...], w_ref[...])


def _filter_body(f_ref, r_ref, w1_ref, b1_ref, w2_ref, b2_ref, o_ref):
    ft = f_ref[...].astype(jnp.bfloat16)
    t = lax.dot_general(ft, w1_ref[...].astype(jnp.bfloat16),
                        (((0,), (0,)), ((), ())),
                        preferred_element_type=jnp.float32) + b1_ref[...]
    t = _ssp(t)
    t = _bdot(t, w2_ref[...]) + b2_ref[...]
    rt = jnp.transpose(r_ref[...])
    for k in range(t.shape[0] // 128):
        o_ref[k * 128:(k + 1) * 128, :] = (
            t[k * 128:(k + 1) * 128, :] * rt[:, k:k + 1]
        ).astype(jnp.bfloat16)


def _out_body(pa_ref, pb_ref, w1_ref, b1_ref, w2_ref, b2_ref, o_ref):
    conv = pa_ref[0] + pb_ref[0]
    t = _ssp(_bdot(conv, w1_ref[...]) + b1_ref[...])
    o_ref[...] = _bdot(t, w2_ref[...]) + b2_ref[...]


def _rep(shape):
    return pl.BlockSpec(shape, lambda i: (0, 0))


_QL = []
for _b in range(D // 32):
    for _k in range(16):
        _QL += [32 * _b + _k, 32 * _b + 16 + _k]


def _pairs_i32(x):
    n, d = x.shape
    return lax.bitcast_convert_type(x.reshape(n, d // 2, 2), jnp.int32)


BPB = 32


def _conv_sc(h, wij, idx_i, idx_j, pvec, parity, total_p):
    mesh = plsc.VectorSubcoreMesh(core_axis_name="c", subcore_axis_name="s")
    cp = pltpu.CompilerParams()
    if "needs_layout_passes" in pltpu.CompilerParams.__dataclass_fields__:
        cp = dataclasses.replace(cp, needs_layout_passes=False)

    @functools.partial(
        pl.kernel,
        mesh=mesh,
        compiler_params=cp,
        out_type=jax.ShapeDtypeStruct((NC, ACC_ROWS, D), jnp.float32),
        scratch_types=(
            [pltpu.VMEM((CHUNK,), jnp.int32)] * (2 * IR)
            + [pltpu.VMEM((CHUNK, D), jnp.float32)] * 2
            + [pltpu.VMEM((CHUNK, D // 2), jnp.int32)] * 2
            + [pltpu.VMEM((CHUNK, D), jnp.float32)]
            + [pltpu.VMEM((CHUNK,), jnp.int32)]
            + [pltpu.VMEM_SHARED((ACC_ROWS, D), jnp.float32)]
            + [pltpu.VMEM((16,), jnp.int32)]
            + [pltpu.SemaphoreType.DMA] * (IR + 5)
        ),
    )
    def k(h_hbm, w_hbm, ii_hbm, ij_hbm, p_hbm, out_hbm,
          ii0, ii1, ii2, ii3, ij0, ij1, ij2, ij3,
          r0b, r1b, w0b, w1b, xb, ab, acc, psm,
          si0, si1, si2, si3, sg0, sg1, ss0, sw0, sw1):
        iiv = (ii0, ii1, ii2, ii3)
        ijv = (ij0, ij1, ij2, ij3)
        rows = (r0b, r1b)
        wv = (w0b, w1b)
        si = (si0, si1, si2, si3)
        sg = (sg0, sg1)
        sw = (sw0, sw1)

        c = lax.axis_index("c")
        s = lax.axis_index("s")

        pltpu.sync_copy(p_hbm, psm)
        p = jnp.max(psm[...])
        def count_par(x):
            b, r = x // BPB, x % BPB
            return (BPB * ((b + 1 - parity) // 2)
                    + jnp.where(b % 2 == parity, r, 0))

        skip = count_par(p // CHUNK)
        cnt_c = jnp.where(c == 0, count_par((p + CHUNK - 1) // CHUNK),
                          total_p - skip)
        goff = jnp.where(c == 0, 0, skip)

        @pl.loop(0, CHUNK)
        def _(r):
            zr = xb.at[r]
            for j in range(0, D, 16):
                zr[pl.ds(j, 16)] = jnp.zeros((16,), jnp.float32)

        row0 = s * SLAB
        for t, nr in ((0, 128), (1, 128), (2, 64)):
            pltpu.sync_copy(xb.at[pl.ds(0, nr)],
                            acc.at[pl.ds(row0 + t * 128, nr)])

        plsc.subcore_barrier()

        def active(slot):
            return s + NS * slot < cnt_c

        def g_of(slot):
            return goff + s + NS * slot

        def ci_of(slot):
            g = g_of(slot)
            return BPB * (parity + 2 * (g // BPB)) + g % BPB

        def issue_idx(slot, r):
            off = ci_of(slot) * CHUNK
            pltpu.async_copy(ii_hbm.at[pl.ds(off, CHUNK)], iiv[r], si[r])
            pltpu.async_copy(ij_hbm.at[pl.ds(off, CHUNK)], ijv[r], si[r])

        def wait_idx(r):
            pltpu.make_async_copy(ii_hbm.at[pl.ds(0, CHUNK)], iiv[r],
                                  si[r]).wait()
            pltpu.make_async_copy(ij_hbm.at[pl.ds(0, CHUNK)], ijv[r],
                                  si[r]).wait()

        def issue_gather(r, b):
            pltpu.async_copy(h_hbm.at[ijv[r]], rows[b], sg[b])

        def wait_gather(r, b):
            pltpu.make_async_copy(h_hbm.at[ijv[r]], rows[b], sg[b]).wait()

        def issue_wij(slot, b):
            off = g_of(slot) * CHUNK
            pltpu.async_copy(w_hbm.at[pl.ds(off, CHUNK)], wv[b], sw[b])

        def wait_wij(b):
            pltpu.make_async_copy(w_hbm.at[pl.ds(0, CHUNK)], wv[b],
                                  sw[b]).wait()

        def wait_scatter():
            pltpu.make_async_copy(xb, acc.at[ab], ss0).wait()

        for u in range(IR):
            @pl.when(active(u))
            def _(u=u):
                issue_idx(u, u)
        for u in range(2):
            @pl.when(active(u))
            def _(u=u):
                wait_idx(u)
                issue_gather(u, u)
                issue_wij(u, u)

        tmax = -(-total_p // NS)
        tmax += (-tmax) % 4

        @pl.loop(0, tmax, step=4)
        def _(t):
            for u in range(4):
                b = u % 2
                r = u % IR

                @pl.when(active(t + u))
                def _(u=u, b=b, r=r):
                    slot = t + u
                    wait_gather(r, b)
                    wait_wij(b)

                    @pl.when(slot >= 1)
                    def _():
                        wait_scatter()

                    coff = c * SPLIT
                    for j in range(0, CHUNK, 16):
                        v = iiv[r][pl.ds(j, 16)] - coff
                        v = jnp.where((v < 0) | (v >= SPLIT), SPLIT, v)
                        ab[pl.ds(j, 16)] = v

                    himask = jnp.full((16,), -65536, jnp.int32)

                    @pl.loop(0, CHUNK)
                    def _(rr):
                        rv = rows[b].at[rr]
                        wr = wv[b].at[rr]
                        xr = xb.at[rr]
                        for j in range(0, D // 2, 16):
                            w32 = wr[pl.ds(j, 16)]
                            we = plsc.bitcast(w32 << 16, jnp.float32)
                            wo = plsc.bitcast(w32 & himask, jnp.float32)
                            xr[pl.ds(2 * j, 16)] = rv[pl.ds(2 * j, 16)] * we
                            xr[pl.ds(2 * j + 16, 16)] = (
                                rv[pl.ds(2 * j + 16, 16)] * wo)

                    pltpu.async_copy(xb, acc.at[ab], ss0, add=True)

                    @pl.when(active(slot + 2))
                    def _():
                        wait_idx((u + 2) % IR)
                        issue_gather((u + 2) % IR, b)
                        issue_wij(slot + 2, b)

                    @pl.when(active(slot + IR))
                    def _():
                        issue_idx(slot + IR, r)

        @pl.when(active(0))
        def _():
            wait_scatter()

        plsc.subcore_barrier()

        for t, nr in ((0, 128), (1, 128), (2, 64)):
            pltpu.sync_copy(acc.at[pl.ds(row0 + t * 128, nr)],
                            xb.at[pl.ds(0, nr)])
            pltpu.sync_copy(xb.at[pl.ds(0, nr)],
                            out_hbm.at[c].at[pl.ds(row0 + t * 128, nr)])

    return k(h, wij, idx_i, idx_j, pvec)


def kernel(x, f_ij, idx_i, idx_j, rcut_ij,
           W_in2f, W_f1, b_f1, W_f2, b_f2, W_o1, b_o1, W_o2, b_o2):
    AB = 1000
    EB = 4096

    ql = jnp.array(_QL, dtype=jnp.int32)
    h = pl.pallas_call(
        _h_body,
        grid=(N_ATOMS // AB,),
        in_specs=[pl.BlockSpec((AB, D), lambda i: (i, 0)), _rep((D, D))],
        out_specs=pl.BlockSpec((AB, D), lambda i: (i, 0)),
        out_shape=jax.ShapeDtypeStruct((N_ATOMS, D), jnp.float32),
    )(x, W_in2f)

    fT = f_ij.T
    rcp = rcut_ij.reshape(N_EDGES // 128, 128)
    NBLK = -(-N_EDGES // EB)
    NB_A = -(-NBLK // 2)
    NB_B = NBLK // 2
    TOT_A = 1252
    TOT_B = 1248

    def filter_half(nblocks, par, nrows):
        return pl.pallas_call(
            _filter_body,
            grid=(nblocks,),
            in_specs=[
                pl.BlockSpec((N_RBF, EB), lambda i: (0, 2 * i + par)),
                pl.BlockSpec((EB // 128, 128), lambda i: (2 * i + par, 0)),
                _rep((N_RBF, D)), _rep((1, D)), _rep((D, D)), _rep((1, D)),
            ],
            out_specs=pl.BlockSpec((EB, D), lambda i: (i, 0)),
            out_shape=jax.ShapeDtypeStruct((nrows, D), jnp.bfloat16),
        )(fT, rcp, W_f1, b_f1.reshape(1, -1), W_f2[:, ql],
          b_f2[ql].reshape(1, -1))

    wij_a = filter_half(NB_A, 0, TOT_A * CHUNK)
    wij_b = filter_half(NB_B, 1, TOT_B * CHUNK)

    p = jnp.sum((idx_i < SPLIT).astype(jnp.int32)).astype(jnp.int32)
    pvec = jnp.full((16,), p, dtype=jnp.int32)
    parts_a = _conv_sc(h, _pairs_i32(wij_a), idx_i, idx_j, pvec, 0, TOT_A)
    parts_b = _conv_sc(h, _pairs_i32(wij_b), idx_i, idx_j, pvec, 1, TOT_B)

    NB0 = SPLIT // AB
    pspec = pl.BlockSpec((1, AB, D), lambda i: (i // NB0, i % NB0, 0))
    out = pl.pallas_call(
        _out_body,
        grid=(N_ATOMS // AB,),
        in_specs=[
            pspec, pspec,
            _rep((D, D)), _rep((1, D)), _rep((D, D)), _rep((1, D)),
        ],
        out_specs=pl.BlockSpec((AB, D), lambda i: (i, 0)),
        out_shape=jax.ShapeDtypeStruct((N_ATOMS, D), jnp.float32),
    )(parts_a, parts_b, W_o1, b_o1.reshape(1, -1), W_o2, b_o2.reshape(1, -1))

    return out


# speedup vs baseline: 4.5423x; 4.5423x over previous
import dataclasses
import functools

import jax
import jax.numpy as jnp
from jax import lax
from jax.experimental import pallas as pl
from jax.experimental.pallas import tpu as pltpu
from jax.experimental.pallas import tpu_sc as plsc

N_ATOMS = 10000
N_EDGES = 320000
D = 128
N_RBF = 20

NC = 2
NS = 16
NW = NC * NS
CHUNK = 128
NCH_TOTAL = N_EDGES // CHUNK
SPLIT = N_ATOMS // NC
ACC_ROWS = 5120
SLAB = ACC_ROWS // NS
TMAX = 160
IR = 4


_LOG2 = 0.6931471805599453


def _ssp(t):
    return jnp.maximum(t, 0.0) + jnp.log(0.5 * (1.0 + jnp.exp(-jnp.abs(t))))


def _bdot(a, b):
    return jnp.dot(a.astype(jnp.bfloat16), b.astype(jnp.bfloat16),
                   preferred_element_type=jnp.float32)


def _h_body(x_ref, w_ref, o_ref):
    o_ref[...] = _bdot(x_ref[...], w_ref[...])


def _filter_body(f_ref, r_ref, w1_ref, b1_ref, w2_ref, b2_ref, o_ref):
    ft = f_ref[...].astype(jnp.bfloat16)
    t = lax.dot_general(ft, w1_ref[...].astype(jnp.bfloat16),
                        (((0,), (0,)), ((), ())),
                        preferred_element_type=jnp.float32) + b1_ref[...]
    t = _ssp(t)
    t = _bdot(t, w2_ref[...]) + b2_ref[...]
    rt = jnp.transpose(r_ref[...])
    for k in range(t.shape[0] // 128):
        o_ref[k * 128:(k + 1) * 128, :] = (
            t[k * 128:(k + 1) * 128, :] * rt[:, k:k + 1])


def _out_body(pa_ref, pb_ref, w1_ref, b1_ref, w2_ref, b2_ref, o_ref):
    conv = pa_ref[0] + pb_ref[0]
    t = _ssp(_bdot(conv, w1_ref[...]) + b1_ref[...])
    o_ref[...] = _bdot(t, w2_ref[...]) + b2_ref[...]


def _rep(shape):
    return pl.BlockSpec(shape, lambda i: (0, 0))


_QL = []
for _b in range(D // 32):
    for _k in range(16):
        _QL += [32 * _b + _k, 32 * _b + 16 + _k]


def _pairs_i32(x):
    n, d = x.shape
    return lax.bitcast_convert_type(x.reshape(n, d // 2, 2), jnp.int32)


BPB = 32


def _conv_sc(h, wij, idx_i, idx_j, pvec, parity, total_p):
    mesh = plsc.VectorSubcoreMesh(core_axis_name="c", subcore_axis_name="s")
    cp = pltpu.CompilerParams()
    if "needs_layout_passes" in pltpu.CompilerParams.__dataclass_fields__:
        cp = dataclasses.replace(cp, needs_layout_passes=False)

    @functools.partial(
        pl.kernel,
        mesh=mesh,
        compiler_params=cp,
        out_type=jax.ShapeDtypeStruct((NC, ACC_ROWS, D), jnp.float32),
        scratch_types=(
            [pltpu.VMEM((CHUNK,), jnp.int32)] * (2 * IR)
            + [pltpu.VMEM((CHUNK, D), jnp.float32)] * 2
            + [pltpu.VMEM((CHUNK, D), jnp.float32)] * 2
            + [pltpu.VMEM((CHUNK, D), jnp.float32)]
            + [pltpu.VMEM((CHUNK,), jnp.int32)]
            + [pltpu.VMEM_SHARED((ACC_ROWS, D), jnp.float32)]
            + [pltpu.VMEM((16,), jnp.int32)]
            + [pltpu.SemaphoreType.DMA] * (IR + 5)
        ),
    )
    def k(h_hbm, w_hbm, ii_hbm, ij_hbm, p_hbm, out_hbm,
          ii0, ii1, ii2, ii3, ij0, ij1, ij2, ij3,
          r0b, r1b, w0b, w1b, xb, ab, acc, psm,
          si0, si1, si2, si3, sg0, sg1, ss0, sw0, sw1):
        iiv = (ii0, ii1, ii2, ii3)
        ijv = (ij0, ij1, ij2, ij3)
        rows = (r0b, r1b)
        wv = (w0b, w1b)
        si = (si0, si1, si2, si3)
        sg = (sg0, sg1)
        sw = (sw0, sw1)

        c = lax.axis_index("c")
        s = lax.axis_index("s")

        pltpu.sync_copy(p_hbm, psm)
        p = jnp.max(psm[...])
        def count_par(x):
            b, r = x // BPB, x % BPB
            return (BPB * ((b + 1 - parity) // 2)
                    + jnp.where(b % 2 == parity, r, 0))

        skip = count_par(p // CHUNK)
        cnt_c = jnp.where(c == 0, count_par((p + CHUNK - 1) // CHUNK),
                          total_p - skip)
        goff = jnp.where(c == 0, 0, skip)

        @pl.loop(0, CHUNK)
        def _(r):
            zr = xb.at[r]
            for j in range(0, D, 16):
                zr[pl.ds(j, 16)] = jnp.zeros((16,), jnp.float32)

        row0 = s * SLAB
        for t, nr in ((0, 128), (1, 128), (2, 64)):
            pltpu.sync_copy(xb.at[pl.ds(0, nr)],
                            acc.at[pl.ds(row0 + t * 128, nr)])

        plsc.subcore_barrier()

        def active(slot):
            return s + NS * slot < cnt_c

        def g_of(slot):
            return goff + s + NS * slot

        def ci_of(slot):
            g = g_of(slot)
            return BPB * (parity + 2 * (g // BPB)) + g % BPB

        def issue_idx(slot, r):
            off = ci_of(slot) * CHUNK
            pltpu.async_copy(ii_hbm.at[pl.ds(off, CHUNK)], iiv[r], si[r])
            pltpu.async_copy(ij_hbm.at[pl.ds(off, CHUNK)], ijv[r], si[r])

        def wait_idx(r):
            pltpu.make_async_copy(ii_hbm.at[pl.ds(0, CHUNK)], iiv[r],
                                  si[r]).wait()
            pltpu.make_async_copy(ij_hbm.at[pl.ds(0, CHUNK)], ijv[r],
                                  si[r]).wait()

        def issue_gather(r, b):
            pltpu.async_copy(h_hbm.at[ijv[r]], rows[b], sg[b])

        def wait_gather(r, b):
            pltpu.make_async_copy(h_hbm.at[ijv[r]], rows[b], sg[b]).wait()

        def issue_wij(slot, b):
            off = g_of(slot) * CHUNK
            pltpu.async_copy(w_hbm.at[pl.ds(off, CHUNK)], wv[b], sw[b])

        def wait_wij(b):
            pltpu.make_async_copy(w_hbm.at[pl.ds(0, CHUNK)], wv[b],
                                  sw[b]).wait()

        def wait_scatter():
            pltpu.make_async_copy(xb, acc.at[ab], ss0).wait()

        for u in range(IR):
            @pl.when(active(u))
            def _(u=u):
                issue_idx(u, u)
        for u in range(2):
            @pl.when(active(u))
            def _(u=u):
                wait_idx(u)
                issue_gather(u, u)
                issue_wij(u, u)

        tmax = -(-total_p // NS)
        tmax += (-tmax) % 4

        @pl.loop(0, tmax, step=4)
        def _(t):
            for u in range(4):
                b = u % 2
                r = u % IR

                @pl.when(active(t + u))
                def _(u=u, b=b, r=r):
                    slot = t + u
                    wait_gather(r, b)
                    wait_wij(b)

                    @pl.when(slot >= 1)
                    def _():
                        wait_scatter()

                    coff = c * SPLIT
                    for j in range(0, CHUNK, 16):
                        v = iiv[r][pl.ds(j, 16)] - coff
                        v = jnp.where((v < 0) | (v >= SPLIT), SPLIT, v)
                        ab[pl.ds(j, 16)] = v

                    @pl.loop(0, CHUNK)
                    def _(rr):
                        rv = rows[b].at[rr]
                        wr = wv[b].at[rr]
                        xr = xb.at[rr]
                        for j in range(0, D, 16):
                            xr[pl.ds(j, 16)] = (
                                rv[pl.ds(j, 16)] * wr[pl.ds(j, 16)])

                    pltpu.async_copy(xb, acc.at[ab], ss0, add=True)

                    @pl.when(active(slot + 2))
                    def _():
                        wait_idx((u + 2) % IR)
                        issue_gather((u + 2) % IR, b)
                        issue_wij(slot + 2, b)

                    @pl.when(active(slot + IR))
                    def _():
                        issue_idx(slot + IR, r)

        @pl.when(active(0))
        def _():
            wait_scatter()

        plsc.subcore_barrier()

        for t, nr in ((0, 128), (1, 128), (2, 64)):
            pltpu.sync_copy(acc.at[pl.ds(row0 + t * 128, nr)],
                            xb.at[pl.ds(0, nr)])
            pltpu.sync_copy(xb.at[pl.ds(0, nr)],
                            out_hbm.at[c].at[pl.ds(row0 + t * 128, nr)])

    return k(h, wij, idx_i, idx_j, pvec)


def kernel(x, f_ij, idx_i, idx_j, rcut_ij,
           W_in2f, W_f1, b_f1, W_f2, b_f2, W_o1, b_o1, W_o2, b_o2):
    AB = 1000
    EB = 4096

    h = pl.pallas_call(
        _h_body,
        grid=(N_ATOMS // AB,),
        in_specs=[pl.BlockSpec((AB, D), lambda i: (i, 0)), _rep((D, D))],
        out_specs=pl.BlockSpec((AB, D), lambda i: (i, 0)),
        out_shape=jax.ShapeDtypeStruct((N_ATOMS, D), jnp.float32),
    )(x, W_in2f)

    fT = f_ij.T
    rcp = rcut_ij.reshape(N_EDGES // 128, 128)
    NBLK = -(-N_EDGES // EB)
    NB_A = -(-NBLK // 2)
    NB_B = NBLK // 2
    TOT_A = 1252
    TOT_B = 1248

    def filter_half(nblocks, par, nrows):
        return pl.pallas_call(
            _filter_body,
            grid=(nblocks,),
            in_specs=[
                pl.BlockSpec((N_RBF, EB), lambda i: (0, 2 * i + par)),
                pl.BlockSpec((EB // 128, 128), lambda i: (2 * i + par, 0)),
                _rep((N_RBF, D)), _rep((1, D)), _rep((D, D)), _rep((1, D)),
            ],
            out_specs=pl.BlockSpec((EB, D), lambda i: (i, 0)),
            out_shape=jax.ShapeDtypeStruct((nrows, D), jnp.float32),
        )(fT, rcp, W_f1, b_f1.reshape(1, -1), W_f2, b_f2.reshape(1, -1))

    wij_a = filter_half(NB_A, 0, TOT_A * CHUNK)
    wij_b = filter_half(NB_B, 1, TOT_B * CHUNK)

    p = jnp.sum((idx_i < SPLIT).astype(jnp.int32)).astype(jnp.int32)
    pvec = jnp.full((16,), p, dtype=jnp.int32)
    parts_a = _conv_sc(h, wij_a, idx_i, idx_j, pvec, 0, TOT_A)
    parts_b = _conv_sc(h, wij_b, idx_i, idx_j, pvec, 1, TOT_B)

    NB0 = SPLIT // AB
    pspec = pl.BlockSpec((1, AB, D), lambda i: (i // NB0, i % NB0, 0))
    out = pl.pallas_call(
        _out_body,
        grid=(N_ATOMS // AB,),
        in_specs=[
            pspec, pspec,
            _rep((D, D)), _rep((1, D)), _rep((D, D)), _rep((1, D)),
        ],
        out_specs=pl.BlockSpec((AB, D), lambda i: (i, 0)),
        out_shape=jax.ShapeDtypeStruct((N_ATOMS, D), jnp.float32),
    )(parts_a, parts_b, W_o1, b_o1.reshape(1, -1), W_o2, b_o2.reshape(1, -1))

    return out
